# Initial kernel scaffold; baseline (speedup 1.0000x reference)
import jax
import jax.numpy as jnp
from jax.experimental import pallas as pl


def kernel(features, train_mat_edges, x, W, b):
    raise NotImplementedError("write your pallas kernel here")


# speedup vs baseline: 29.5389x; 29.5389x over previous
import functools

import jax
import jax.numpy as jnp
from jax import lax
from jax.experimental import pallas as pl
from jax.experimental.pallas import tpu as pltpu
from jax.experimental.pallas import tpu_sc as plsc

_N = 10000
_NPAD = 10240
_D = 16
_F = 128
_E = 320000
_CH = 128
_NCHUNK = 160
_EPAD = 16 * _NCHUNK * _CH
_K = 8
_NGRP = _NCHUNK // _K
_ROWS_PER_SUB = _NPAD // 16
_XCH = 26

_mesh = plsc.VectorSubcoreMesh(core_axis_name="c", subcore_axis_name="s")


@functools.partial(
    pl.kernel,
    out_type=jax.ShapeDtypeStruct((32 * _XCH * 128, _D), jnp.float32),
    mesh=_mesh,
    compiler_params=pltpu.CompilerParams(use_tc_tiling_on_sc=False,
                                         needs_layout_passes=False),
    scratch_types=[
        pltpu.VMEM((_NCHUNK, _CH), jnp.int32),
        pltpu.VMEM((_NCHUNK, _CH), jnp.int32),
        pltpu.VMEM((_K, _CH, _D), jnp.float32),
        pltpu.VMEM((_K, _CH, _D), jnp.float32),
        pltpu.VMEM((_ROWS_PER_SUB, _D), jnp.float32),
        pltpu.VMEM((_ROWS_PER_SUB, _D), jnp.float32),
        pltpu.VMEM((_ROWS_PER_SUB,), jnp.float32),
        pltpu.VMEM((_ROWS_PER_SUB,), jnp.float32),
        pltpu.VMEM((_XCH, 128), jnp.int32),
        pltpu.VMEM((_D,), jnp.float32),
        pltpu.VMEM((_CH,), jnp.float32),
        pltpu.VMEM((128, _D), jnp.float32),
        pltpu.VMEM((128, _D), jnp.float32),
        pltpu.VMEM_SHARED((_NPAD, _D), jnp.float32),
        pltpu.VMEM_SHARED((_NPAD, _D), jnp.float32),
        pltpu.VMEM_SHARED((_NPAD,), jnp.float32),
        pltpu.SemaphoreType.DMA,
        pltpu.SemaphoreType.DMA,
        pltpu.SemaphoreType.DMA,
        pltpu.SemaphoreType.DMA,
        pltpu.SemaphoreType.DMA,
        pltpu.SemaphoreType.DMA,
    ],
)
def _sc_mega(xw_hbm, row_hbm, col_hbm, zeros_hbm, zeros1_hbm, ones_hbm,
             bvec_hbm, x_hbm, out_hbm,
             row_v, col_v, buf_a, buf_b, nod_v, y_v, deg_v, dinv_v, x_v,
             b_v, ones_v, buf0, buf1, y_sh, s_sh, deg_sh,
             gsem_a, gsem_b, ssem_a, ssem_b, w0, w1):
    c = lax.axis_index("c")
    s = lax.axis_index("s")
    tid = c * 16 + s
    base = tid * (_XCH * 128)
    sl = pl.ds(s * _ROWS_PER_SUB, _ROWS_PER_SUB)

    pltpu.sync_copy(x_hbm.at[tid], x_v)
    pltpu.sync_copy(row_hbm.at[s], row_v)
    pltpu.sync_copy(col_hbm.at[s], col_v)
    pltpu.sync_copy(zeros_hbm.at[sl], s_sh.at[sl])
    pltpu.sync_copy(zeros1_hbm.at[sl], deg_sh.at[sl])
    pltpu.sync_copy(bvec_hbm, b_v)
    pltpu.sync_copy(ones_hbm, ones_v)
    pltpu.sync_copy(xw_hbm.at[sl], nod_v)
    plsc.subcore_barrier()

    def dgrp(g, carry):
        for i in range(_K):
            pltpu.async_copy(ones_v, deg_sh.at[col_v.at[g * _K + i]], ssem_a,
                             add=True)
        for i in range(_K):
            pltpu.make_async_copy(ones_v, deg_sh.at[col_v.at[g * _K + i]],
                                  ssem_a).wait()
        return carry

    lax.fori_loop(0, _NGRP, dgrp, 0)
    plsc.subcore_barrier()

    pltpu.sync_copy(deg_sh.at[sl], deg_v)

    def dchunk(k, carry):
        d = deg_v[pl.ds(k * 16, 16)] + 1.0
        xi = 0x5F3759DF - lax.shift_right_logical(
            plsc.bitcast(d, jnp.int32), 1)
        xf = plsc.bitcast(xi, jnp.float32)
        for _ in range(3):
            xf = xf * (1.5 - 0.5 * d * xf * xf)
        dinv_v[pl.ds(k * 16, 16)] = xf
        return carry

    lax.fori_loop(0, _ROWS_PER_SUB // 16, dchunk, 0)

    def yrow(r, carry):
        bc = plsc.load_gather(dinv_v, [jnp.full((16,), 0, jnp.int32) + r])
        y_v[r, :] = nod_v[r, :] * bc
        return carry

    lax.fori_loop(0, _ROWS_PER_SUB, yrow, 0)
    pltpu.sync_copy(y_v, y_sh.at[sl])
    plsc.subcore_barrier()

    def gather_grp(g, buf, sem):
        for i in range(_K):
            pltpu.async_copy(y_sh.at[row_v.at[g * _K + i]], buf.at[i], sem)

    def wait_gather(buf, sem):
        for i in range(_K):
            pltpu.make_async_copy(y_sh.at[row_v.at[0]], buf.at[i], sem).wait()

    def scatter_grp(g, buf, sem):
        for i in range(_K):
            pltpu.async_copy(buf.at[i], s_sh.at[col_v.at[g * _K + i]], sem,
                             add=True)

    def wait_scatter(buf, sem):
        for i in range(_K):
            pltpu.make_async_copy(buf.at[i], s_sh.at[col_v.at[0]], sem).wait()

    gather_grp(0, buf_a, gsem_a)

    def body(m, carry):
        ga = 2 * m
        gb = 2 * m + 1
        wait_gather(buf_a, gsem_a)
        gather_grp(gb, buf_b, gsem_b)
        scatter_grp(ga, buf_a, ssem_a)
        wait_gather(buf_b, gsem_b)
        wait_scatter(buf_a, ssem_a)

        @pl.when(m < _NGRP // 2 - 1)
        def _():
            gather_grp(ga + 2, buf_a, gsem_a)

        scatter_grp(gb, buf_b, ssem_b)
        wait_scatter(buf_b, ssem_b)
        return carry

    lax.fori_loop(0, _NGRP // 2, body, 0)
    plsc.subcore_barrier()

    pltpu.sync_copy(s_sh.at[sl], nod_v)
    bv = b_v[...]

    def embrow(r, carry):
        bc = plsc.load_gather(dinv_v, [jnp.full((16,), 0, jnp.int32) + r])
        nod_v[r, :] = bc * (nod_v[r, :] + y_v[r, :]) + bv
        return carry

    lax.fori_loop(0, _ROWS_PER_SUB, embrow, 0)
    pltpu.sync_copy(nod_v, s_sh.at[sl])
    plsc.subcore_barrier()

    def xbody(m, carry):
        ja = 2 * m
        jb = 2 * m + 1
        pltpu.make_async_copy(s_sh.at[x_v.at[0]], buf0, gsem_a).wait()
        pltpu.async_copy(s_sh.at[x_v.at[jb]], buf1, gsem_b)
        pltpu.async_copy(buf0, out_hbm.at[pl.ds(base + ja * 128, 128)], w0)
        pltpu.make_async_copy(s_sh.at[x_v.at[0]], buf1, gsem_b).wait()
        pltpu.make_async_copy(buf0, out_hbm.at[pl.ds(base, 128)], w0).wait()

        @pl.when(m < _XCH // 2 - 1)
        def _():
            pltpu.async_copy(s_sh.at[x_v.at[ja + 2]], buf0, gsem_a)

        pltpu.async_copy(buf1, out_hbm.at[pl.ds(base + jb * 128, 128)], w1)
        pltpu.make_async_copy(buf1, out_hbm.at[pl.ds(base, 128)], w1).wait()
        return carry

    pltpu.async_copy(s_sh.at[x_v.at[0]], buf0, gsem_a)
    lax.fori_loop(0, _XCH // 2, xbody, 0)


def _tc_xw_body(feat_ref, w_ref, xw_ref):
    xw_ref[...] = jnp.dot(feat_ref[...], w_ref[...],
                          preferred_element_type=jnp.float32)


def kernel(features, train_mat_edges, x, W, b):
    f32 = jnp.float32
    row = train_mat_edges[0]
    col = train_mat_edges[1]
    pad = _EPAD - _E
    rowp = jnp.concatenate(
        [row, jnp.full((pad,), _N, jnp.int32)]).reshape(16, _NCHUNK, _CH)
    colp = jnp.concatenate(
        [col, jnp.full((pad,), _NPAD - 1, jnp.int32)]).reshape(
            16, _NCHUNK, _CH)
    featp = jnp.concatenate(
        [features, jnp.zeros((_NPAD - _N, _F), f32)], axis=0)
    zeros = jnp.zeros((_NPAD, _D), f32)
    zeros1 = jnp.zeros((_NPAD,), f32)
    ones1 = jnp.ones((_CH,), f32)
    x3 = x.reshape(32, _XCH, 128)

    xw = pl.pallas_call(
        _tc_xw_body,
        out_shape=jax.ShapeDtypeStruct((_NPAD, _D), f32),
    )(featp, W)
    outflat = _sc_mega(xw, rowp, colp, zeros, zeros1, ones1, b, x3)
    return outflat.reshape(x.shape[0], x.shape[1], _D)


# speedup vs baseline: 32.1734x; 1.0892x over previous
import functools

import jax
import jax.numpy as jnp
from jax import lax
from jax.experimental import pallas as pl
from jax.experimental.pallas import tpu as pltpu
from jax.experimental.pallas import tpu_sc as plsc

_N = 10000
_NPAD = 10240
_D = 16
_F = 128
_E = 320000
_NTILES = 32
_CH = 128
_NCHUNK = 80
_EPAD = _NTILES * _NCHUNK * _CH
_K = 8
_NGRP = _NCHUNK // _K
_ROWS_PER_SUB = _NPAD // 16
_XCH = 26

_mesh = plsc.VectorSubcoreMesh(core_axis_name="c", subcore_axis_name="s")
_params = pltpu.CompilerParams(use_tc_tiling_on_sc=False,
                               needs_layout_passes=False)


def _zero_vmem(buf, nrows):
    def z(k, carry):
        buf[pl.ds(k * 16, 16)] = jnp.zeros((16,), jnp.float32)
        return carry

    lax.fori_loop(0, nrows // 16, z, 0)


def _newton_dinv(deg_v, dinv_v):
    def dchunk(k, carry):
        d = deg_v[pl.ds(k * 16, 16)] + 1.0
        xi = 0x5F3759DF - lax.shift_right_logical(
            plsc.bitcast(d, jnp.int32), 1)
        xf = plsc.bitcast(xi, jnp.float32)
        for _ in range(3):
            xf = xf * (1.5 - 0.5 * d * xf * xf)
        dinv_v[pl.ds(k * 16, 16)] = xf
        return carry

    lax.fori_loop(0, _ROWS_PER_SUB // 16, dchunk, 0)


def _bcast(dinv_v, r):
    return plsc.load_gather(dinv_v, [jnp.full((16,), 0, jnp.int32) + r])


@functools.partial(
    pl.kernel,
    out_type=jax.ShapeDtypeStruct((2, _NPAD), jnp.float32),
    mesh=_mesh,
    compiler_params=_params,
    scratch_types=[
        pltpu.VMEM((_NCHUNK, _CH), jnp.int32),
        pltpu.VMEM((_CH,), jnp.float32),
        pltpu.VMEM((_ROWS_PER_SUB,), jnp.float32),
        pltpu.VMEM_SHARED((_NPAD,), jnp.float32),
        pltpu.SemaphoreType.DMA,
    ],
)
def _sc_deg(col_hbm, degp_hbm, col_v, ones_v, zbuf, deg_sh, ssem):
    c = lax.axis_index("c")
    s = lax.axis_index("s")
    tid = c * 16 + s
    sl = pl.ds(s * _ROWS_PER_SUB, _ROWS_PER_SUB)
    _zero_vmem(zbuf, _ROWS_PER_SUB)

    def o(k, carry):
        ones_v[pl.ds(k * 16, 16)] = jnp.ones((16,), jnp.float32)
        return carry

    lax.fori_loop(0, _CH // 16, o, 0)
    pltpu.sync_copy(zbuf, deg_sh.at[sl])
    pltpu.sync_copy(col_hbm.at[tid], col_v)
    plsc.subcore_barrier()

    def grp(g, carry):
        for i in range(_K):
            pltpu.async_copy(ones_v, deg_sh.at[col_v.at[g * _K + i]], ssem,
                             add=True)
        for i in range(_K):
            pltpu.make_async_copy(ones_v, deg_sh.at[col_v.at[g * _K + i]],
                                  ssem).wait()
        return carry

    lax.fori_loop(0, _NGRP, grp, 0)
    plsc.subcore_barrier()
    pltpu.sync_copy(deg_sh.at[sl], degp_hbm.at[c, sl])


@functools.partial(
    pl.kernel,
    out_type=jax.ShapeDtypeStruct((2, _NPAD, _D), jnp.float32),
    mesh=_mesh,
    compiler_params=_params,
    scratch_types=[
        pltpu.VMEM((_NCHUNK, _CH), jnp.int32),
        pltpu.VMEM((_NCHUNK, _CH), jnp.int32),
        pltpu.VMEM((_K, _CH, _D), jnp.float32),
        pltpu.VMEM((_K, _CH, _D), jnp.float32),
        pltpu.VMEM((_ROWS_PER_SUB, _D), jnp.float32),
        pltpu.VMEM((_ROWS_PER_SUB, _D), jnp.float32),
        pltpu.VMEM((_ROWS_PER_SUB,), jnp.float32),
        pltpu.VMEM((_ROWS_PER_SUB,), jnp.float32),
        pltpu.VMEM_SHARED((_NPAD, _D), jnp.float32),
        pltpu.VMEM_SHARED((_NPAD, _D), jnp.float32),
        pltpu.SemaphoreType.DMA,
        pltpu.SemaphoreType.DMA,
        pltpu.SemaphoreType.DMA,
        pltpu.SemaphoreType.DMA,
    ],
)
def _sc_seg(xw_hbm, degp_hbm, row_hbm, col_hbm, sp_hbm,
            row_v, col_v, buf_a, buf_b, nod_v, y_v, deg_v, dinv_v,
            y_sh, s_sh, gsem_a, gsem_b, ssem_a, ssem_b):
    c = lax.axis_index("c")
    s = lax.axis_index("s")
    tid = c * 16 + s
    sl = pl.ds(s * _ROWS_PER_SUB, _ROWS_PER_SUB)
    pltpu.sync_copy(row_hbm.at[tid], row_v)
    pltpu.sync_copy(col_hbm.at[tid], col_v)
    pltpu.sync_copy(xw_hbm.at[sl], nod_v)
    pltpu.sync_copy(degp_hbm.at[0, sl], deg_v)
    pltpu.sync_copy(degp_hbm.at[1, sl], dinv_v)

    def addp(k, carry):
        deg_v[pl.ds(k * 16, 16)] = (deg_v[pl.ds(k * 16, 16)]
                                    + dinv_v[pl.ds(k * 16, 16)])
        return carry

    lax.fori_loop(0, _ROWS_PER_SUB // 16, addp, 0)
    _newton_dinv(deg_v, dinv_v)

    def yrow(r, carry):
        y_v[r, :] = nod_v[r, :] * _bcast(dinv_v, r)
        return carry

    lax.fori_loop(0, _ROWS_PER_SUB, yrow, 0)
    pltpu.sync_copy(y_v, y_sh.at[sl])

    def zrow(r, carry):
        nod_v[r, :] = jnp.zeros((16,), jnp.float32)
        return carry

    lax.fori_loop(0, _ROWS_PER_SUB, zrow, 0)
    pltpu.sync_copy(nod_v, s_sh.at[sl])
    plsc.subcore_barrier()

    def gather_grp(g, buf, sem):
        for i in range(_K):
            pltpu.async_copy(y_sh.at[row_v.at[g * _K + i]], buf.at[i], sem)

    def wait_gather(buf, sem):
        for i in range(_K):
            pltpu.make_async_copy(y_sh.at[row_v.at[0]], buf.at[i], sem).wait()

    def scatter_grp(g, buf, sem):
        for i in range(_K):
            pltpu.async_copy(buf.at[i], s_sh.at[col_v.at[g * _K + i]], sem,
                             add=True)

    def wait_scatter(buf, sem):
        for i in range(_K):
            pltpu.make_async_copy(buf.at[i], s_sh.at[col_v.at[0]], sem).wait()

    gather_grp(0, buf_a, gsem_a)

    def body(m, carry):
        ga = 2 * m
        gb = 2 * m + 1
        wait_gather(buf_a, gsem_a)
        gather_grp(gb, buf_b, gsem_b)
        scatter_grp(ga, buf_a, ssem_a)
        wait_gather(buf_b, gsem_b)
        wait_scatter(buf_a, ssem_a)

        @pl.when(m < _NGRP // 2 - 1)
        def _():
            gather_grp(ga + 2, buf_a, gsem_a)

        scatter_grp(gb, buf_b, ssem_b)
        wait_scatter(buf_b, ssem_b)
        return carry

    lax.fori_loop(0, _NGRP // 2, body, 0)
    plsc.subcore_barrier()
    pltpu.sync_copy(s_sh.at[sl], sp_hbm.at[c, sl])


@functools.partial(
    pl.kernel,
    out_type=jax.ShapeDtypeStruct((_NTILES * _XCH * 128, _D), jnp.float32),
    mesh=_mesh,
    compiler_params=_params,
    scratch_types=[
        pltpu.VMEM((_XCH, 128), jnp.int32),
        pltpu.VMEM((_ROWS_PER_SUB, _D), jnp.float32),
        pltpu.VMEM((_ROWS_PER_SUB, _D), jnp.float32),
        pltpu.VMEM((_ROWS_PER_SUB, _D), jnp.float32),
        pltpu.VMEM((_ROWS_PER_SUB,), jnp.float32),
        pltpu.VMEM((_ROWS_PER_SUB,), jnp.float32),
        pltpu.VMEM((_D,), jnp.float32),
        pltpu.VMEM_SHARED((_NPAD, _D), jnp.float32),
        pltpu.VMEM((128, _D), jnp.float32),
        pltpu.VMEM((128, _D), jnp.float32),
        pltpu.SemaphoreType.DMA,
        pltpu.SemaphoreType.DMA,
        pltpu.SemaphoreType.DMA,
        pltpu.SemaphoreType.DMA,
    ],
)
def _sc_emb_gather(sp_hbm, degp_hbm, xw_hbm, bvec_hbm, x_hbm, out_hbm,
                   x_v, s0_v, s1_v, y_v, deg_v, dinv_v, b_v, emb_sh,
                   buf0, buf1, g0, g1, w0, w1):
    c = lax.axis_index("c")
    s = lax.axis_index("s")
    tid = c * 16 + s
    base = tid * (_XCH * 128)
    sl = pl.ds(s * _ROWS_PER_SUB, _ROWS_PER_SUB)
    pltpu.sync_copy(x_hbm.at[tid], x_v)
    pltpu.sync_copy(sp_hbm.at[0, sl], s0_v)
    pltpu.sync_copy(sp_hbm.at[1, sl], s1_v)
    pltpu.sync_copy(xw_hbm.at[sl], y_v)
    pltpu.sync_copy(degp_hbm.at[0, sl], deg_v)
    pltpu.sync_copy(degp_hbm.at[1, sl], dinv_v)
    pltpu.sync_copy(bvec_hbm, b_v)

    def addp(k, carry):
        deg_v[pl.ds(k * 16, 16)] = (deg_v[pl.ds(k * 16, 16)]
                                    + dinv_v[pl.ds(k * 16, 16)])
        return carry

    lax.fori_loop(0, _ROWS_PER_SUB // 16, addp, 0)
    _newton_dinv(deg_v, dinv_v)
    bv = b_v[...]

    def embrow(r, carry):
        bc = _bcast(dinv_v, r)
        s0_v[r, :] = bc * (s0_v[r, :] + s1_v[r, :] + y_v[r, :] * bc) + bv
        return carry

    lax.fori_loop(0, _ROWS_PER_SUB, embrow, 0)
    pltpu.sync_copy(s0_v, emb_sh.at[sl])
    plsc.subcore_barrier()

    def body(m, carry):
        ja = 2 * m
        jb = 2 * m + 1
        pltpu.make_async_copy(emb_sh.at[x_v.at[0]], buf0, g0).wait()
        pltpu.async_copy(emb_sh.at[x_v.at[jb]], buf1, g1)
        pltpu.async_copy(buf0, out_hbm.at[pl.ds(base + ja * 128, 128)], w0)
        pltpu.make_async_copy(emb_sh.at[x_v.at[0]], buf1, g1).wait()
        pltpu.make_async_copy(buf0, out_hbm.at[pl.ds(base, 128)], w0).wait()

        @pl.when(m < _XCH // 2 - 1)
        def _():
            pltpu.async_copy(emb_sh.at[x_v.at[ja + 2]], buf0, g0)

        pltpu.async_copy(buf1, out_hbm.at[pl.ds(base + jb * 128, 128)], w1)
        pltpu.make_async_copy(buf1, out_hbm.at[pl.ds(base, 128)], w1).wait()
        return carry

    pltpu.async_copy(emb_sh.at[x_v.at[0]], buf0, g0)
    lax.fori_loop(0, _XCH // 2, body, 0)


def _tc_xw_body(feat_ref, w_ref, xw_ref):
    xw_ref[...] = jnp.dot(feat_ref[...], w_ref[...],
                          preferred_element_type=jnp.float32)


def kernel(features, train_mat_edges, x, W, b):
    f32 = jnp.float32
    row = train_mat_edges[0]
    col = train_mat_edges[1]
    pad = _EPAD - _E
    rowp = jnp.concatenate(
        [row, jnp.full((pad,), _N, jnp.int32)]).reshape(_NTILES, _NCHUNK, _CH)
    colp = jnp.concatenate(
        [col, jnp.full((pad,), _NPAD - 1, jnp.int32)]).reshape(
            _NTILES, _NCHUNK, _CH)
    featp = jnp.concatenate(
        [features, jnp.zeros((_NPAD - _N, _F), f32)], axis=0)
    x3 = x.reshape(_NTILES, _XCH, 128)

    xw = pl.pallas_call(
        _tc_xw_body,
        out_shape=jax.ShapeDtypeStruct((_NPAD, _D), f32),
    )(featp, W)
    degp = _sc_deg(colp)
    sp = _sc_seg(xw, degp, rowp, colp)
    outflat = _sc_emb_gather(sp, degp, xw, b, x3)
    return outflat.reshape(x.shape[0], x.shape[1], _D)
